# Initial kernel scaffold; baseline (speedup 1.0000x reference)
import functools

import jax
import jax.numpy as jnp
from jax import lax
from jax.experimental import pallas as pl
from jax.experimental.pallas import tpu as pltpu
from jax.experimental.pallas import tpu_sc as plsc

NNODE = 10000
NP = 10240
NE = 320000
FEAT = 64
NGRP = 64
ROWS_PER_TILE = 640
DEN_ROWS = NP // FEAT
EW = NE // 32
CH = 80
NCHUNK = EW // CH
BLK = 640
GRID = NP // BLK


def _tc0_body(x_ref, w_ref, as_ref, ad_ref, xp_ref, ss_ref, sd_ref):
    xp = jnp.dot(x_ref[...], w_ref[...], preferred_element_type=jnp.float32)
    xp_ref[...] = xp
    ss_ref[...] = jnp.dot(xp, as_ref[...], preferred_element_type=jnp.float32)
    sd_ref[...] = jnp.dot(xp, ad_ref[...], preferred_element_type=jnp.float32)


def _tc_front0(x, w, a_s, a_d):
    return pl.pallas_call(
        _tc0_body,
        grid=(GRID,),
        in_specs=[
            pl.BlockSpec((BLK, 128), lambda i: (i, 0)),
            pl.BlockSpec((128, FEAT), lambda i: (0, 0)),
            pl.BlockSpec((FEAT, 1), lambda i: (0, 0)),
            pl.BlockSpec((FEAT, 1), lambda i: (0, 0)),
        ],
        out_specs=[
            pl.BlockSpec((BLK, FEAT), lambda i: (i, 0)),
            pl.BlockSpec((BLK, 1), lambda i: (i, 0)),
            pl.BlockSpec((BLK, 1), lambda i: (i, 0)),
        ],
        out_shape=[
            jax.ShapeDtypeStruct((NP, FEAT), jnp.float32),
            jax.ShapeDtypeStruct((NP, 1), jnp.float32),
            jax.ShapeDtypeStruct((NP, 1), jnp.float32),
        ],
    )(x, w, a_s, a_d)


def _tcmid_body(p_ref, b_ref, w_ref, as_ref, ad_ref, xp_ref, ss_ref, sd_ref):
    r = jnp.maximum(p_ref[0] + p_ref[1] + b_ref[...], 0.0)
    xp = jnp.dot(r, w_ref[...], preferred_element_type=jnp.float32)
    xp_ref[...] = xp
    ss_ref[...] = jnp.dot(xp, as_ref[...], preferred_element_type=jnp.float32)
    sd_ref[...] = jnp.dot(xp, ad_ref[...], preferred_element_type=jnp.float32)


def _tc_mid(p, b, w_eff, a_s, a_d):
    return pl.pallas_call(
        _tcmid_body,
        grid=(GRID,),
        in_specs=[
            pl.BlockSpec((2, BLK, FEAT), lambda i: (0, i, 0)),
            pl.BlockSpec((1, FEAT), lambda i: (0, 0)),
            pl.BlockSpec((FEAT, FEAT), lambda i: (0, 0)),
            pl.BlockSpec((FEAT, 1), lambda i: (0, 0)),
            pl.BlockSpec((FEAT, 1), lambda i: (0, 0)),
        ],
        out_specs=[
            pl.BlockSpec((BLK, FEAT), lambda i: (i, 0)),
            pl.BlockSpec((BLK, 1), lambda i: (i, 0)),
            pl.BlockSpec((BLK, 1), lambda i: (i, 0)),
        ],
        out_shape=[
            jax.ShapeDtypeStruct((NP, FEAT), jnp.float32),
            jax.ShapeDtypeStruct((NP, 1), jnp.float32),
            jax.ShapeDtypeStruct((NP, 1), jnp.float32),
        ],
    )(p, b, w_eff, a_s, a_d)


def _tcfin_body(p_ref, b_ref, bat_ref, wf_ref, bf_ref, y_ref, acc_ref):
    i = pl.program_id(0)
    r = jnp.maximum(p_ref[0] + p_ref[1] + b_ref[...], 0.0)
    grp = bat_ref[...]
    onehot = jnp.where(
        grp == lax.broadcasted_iota(jnp.int32, (BLK, NGRP), 1), 1.0, 0.0)
    part = lax.dot_general(onehot, r, (((0,), (0,)), ((), ())),
                           preferred_element_type=jnp.float32)

    @pl.when(i == 0)
    def _():
        acc_ref[...] = part

    @pl.when(i > 0)
    def _():
        acc_ref[...] = acc_ref[...] + part

    @pl.when(i == GRID - 1)
    def _():
        y_ref[...] = jnp.dot(acc_ref[...], wf_ref[...],
                             preferred_element_type=jnp.float32) + bf_ref[...]


def _tc_fin(p, b, batch2, wf_eff, bf):
    return pl.pallas_call(
        _tcfin_body,
        grid=(GRID,),
        in_specs=[
            pl.BlockSpec((2, BLK, FEAT), lambda i: (0, i, 0)),
            pl.BlockSpec((1, FEAT), lambda i: (0, 0)),
            pl.BlockSpec((BLK, 1), lambda i: (i, 0)),
            pl.BlockSpec((FEAT, 1), lambda i: (0, 0)),
            pl.BlockSpec((1, 1), lambda i: (0, 0)),
        ],
        out_specs=pl.BlockSpec((NGRP, 1), lambda i: (0, 0)),
        out_shape=jax.ShapeDtypeStruct((NGRP, 1), jnp.float32),
        scratch_shapes=[pltpu.VMEM((NGRP, NGRP), jnp.float32)],
    )(p, b, batch2, wf_eff, bf)


_MESH = plsc.VectorSubcoreMesh(core_axis_name="c", subcore_axis_name="s")
_I32 = jnp.int32
_F32 = jnp.float32


def _sc_edge_w(src_flat, dst2, ss, sd):
    @functools.partial(
        pl.kernel,
        mesh=_MESH,
        out_type=[
            jax.ShapeDtypeStruct((NE,), _F32),
            jax.ShapeDtypeStruct((32, DEN_ROWS, FEAT), _F32),
        ],
        scratch_types=[
            pltpu.VMEM((EW,), _I32),
            pltpu.VMEM((NCHUNK, CH), _I32),
            pltpu.VMEM((NP,), _F32),
            pltpu.VMEM((NP,), _F32),
            pltpu.VMEM((EW,), _F32),
            pltpu.VMEM((DEN_ROWS, FEAT), _F32),
        ],
    )
    def k(src_hbm, dst_hbm, ss_hbm, sd_hbm, w_hbm, den_hbm,
          src_t, dst_c, ss_t, sd_t, w_t, den_t):
        cax = lax.axis_index("c")
        sax = lax.axis_index("s")
        wid = cax * 16 + sax
        rbase = wid * NCHUNK

        pltpu.sync_copy(src_hbm.at[pl.ds(wid * EW, EW)], src_t)
        pltpu.sync_copy(dst_hbm.at[pl.ds(rbase, NCHUNK), :], dst_c)
        pltpu.sync_copy(ss_hbm, ss_t)
        pltpu.sync_copy(sd_hbm, sd_t)

        zero16 = jnp.zeros((16,), _F32)

        def zden(i, _):
            for q in range(4):
                den_t[i, pl.ds(q * 16, 16)] = zero16
            return 0

        lax.fori_loop(0, DEN_ROWS, zden, 0)

        neg = jnp.full((16,), -3.0e38, _F32)

        def mx(i, carry):
            a, bb = carry
            sl = pl.ds(i * 16, 16)
            return (jnp.maximum(a, ss_t[sl]), jnp.maximum(bb, sd_t[sl]))

        mss, msd = lax.fori_loop(0, NP // 16, mx, (neg, neg))
        cval = jnp.maximum(jnp.max(mss) + jnp.max(msd), 0.0)
        cvec = zero16 + cval

        c63 = jnp.full((16,), 63, _I32)

        def erow(r, _):
            for q in range(CH // 16):
                fl = pl.ds(r * CH + q * 16, 16)
                sl = pl.ds(q * 16, 16)
                si = src_t[fl]
                di = dst_c[r, sl]
                e = (plsc.load_gather(ss_t, [si])
                     + plsc.load_gather(sd_t, [di]))
                e = jnp.where(e > 0.0, e, 0.2 * e)
                w = jnp.exp(e - cvec)
                w_t[fl] = w
                plsc.addupdate_scatter(
                    den_t,
                    [lax.shift_right_logical(di, 6),
                     lax.bitwise_and(di, c63)],
                    w)
            return 0

        lax.fori_loop(0, NCHUNK, erow, 0)

        pltpu.sync_copy(w_t, w_hbm.at[pl.ds(wid * EW, EW)])
        pltpu.sync_copy(den_t, den_hbm.at[wid])

    return k(src_flat, dst2, ss, sd)


def _sc_aggregate(src_flat, dst2, w, den, xp):
    @functools.partial(
        pl.kernel,
        mesh=_MESH,
        out_type=jax.ShapeDtypeStruct((2, NP, FEAT), _F32),
        scratch_types=[
            pltpu.VMEM((EW,), _I32),
            pltpu.VMEM((NCHUNK, CH), _I32),
            pltpu.VMEM((EW,), _F32),
            pltpu.VMEM((DEN_ROWS, FEAT), _F32),
            pltpu.VMEM((32, 10, FEAT), _F32),
            pltpu.VMEM((10, FEAT), _F32),
            pltpu.VMEM((CH,), _F32),
            pltpu.VMEM((CH, FEAT), _F32),
            pltpu.VMEM_SHARED((NP, FEAT), _F32),
            pltpu.VMEM_SHARED((NP, FEAT), _F32),
            pltpu.VMEM_SHARED((DEN_ROWS, FEAT), _F32),
        ],
    )
    def k(src_hbm, dst_hbm, w_hbm, den_hbm, xp_hbm, p_hbm,
          src_t, dst_c, w_t, rden_t, dall, racc, alpha_t, rows,
          xp_s, out_s, rden_s):
        cax = lax.axis_index("c")
        sax = lax.axis_index("s")
        wid = cax * 16 + sax
        rbase = wid * NCHUNK
        nr = sax * 10
        nb = sax * ROWS_PER_TILE

        pltpu.sync_copy(src_hbm.at[pl.ds(wid * EW, EW)], src_t)
        pltpu.sync_copy(dst_hbm.at[pl.ds(rbase, NCHUNK), :], dst_c)
        pltpu.sync_copy(w_hbm.at[pl.ds(wid * EW, EW)], w_t)

        pltpu.sync_copy(den_hbm.at[:, pl.ds(nr, 10), :], dall)

        def rinv(i, _):
            rr = i // 4
            qq = lax.rem(i, 4)
            sl = pl.ds(qq * 16, 16)
            v = jnp.zeros((16,), _F32)
            for t in range(32):
                v = v + dall[t, rr, sl]
            racc[rr, sl] = jnp.where(v > 0.0, 1.0 / v, 0.0)
            return 0

        lax.fori_loop(0, 40, rinv, 0)
        pltpu.sync_copy(racc, rden_s.at[pl.ds(nr, 10), :])

        pltpu.sync_copy(xp_hbm.at[pl.ds(nb, ROWS_PER_TILE), :],
                        xp_s.at[pl.ds(nb, ROWS_PER_TILE), :])

        zero16 = jnp.zeros((16,), _F32)

        def zrow(j, _):
            for q in range(4):
                rows[j, pl.ds(q * 16, 16)] = zero16
            return 0

        lax.fori_loop(0, CH, zrow, 0)
        for rep in range(ROWS_PER_TILE // CH):
            pltpu.sync_copy(rows, out_s.at[pl.ds(nb + rep * CH, CH), :])

        plsc.subcore_barrier()
        pltpu.sync_copy(rden_s, rden_t)

        c63 = jnp.full((16,), 63, _I32)

        def chunk(ci, _):
            eb = ci * CH

            def av(i, _):
                fl = pl.ds(eb + i * 16, 16)
                sl = pl.ds(i * 16, 16)
                di = dst_c[ci, sl]
                rv = plsc.load_gather(
                    rden_t,
                    [lax.shift_right_logical(di, 6),
                     lax.bitwise_and(di, c63)])
                alpha_t[sl] = w_t[fl] * rv
                return 0

            lax.fori_loop(0, CH // 16, av, 0)

            pltpu.sync_copy(xp_s.at[src_t.at[pl.ds(eb, CH)]], rows)

            zero16i = jnp.zeros((16,), _I32)

            def scale(j, _):
                avec = plsc.load_gather(alpha_t, [zero16i + j])
                for q in range(4):
                    sl = pl.ds(q * 16, 16)
                    rows[j, sl] = rows[j, sl] * avec
                return 0

            lax.fori_loop(0, CH, scale, 0)

            pltpu.sync_copy(rows, out_s.at[dst_c.at[ci]], add=True)
            return 0

        lax.fori_loop(0, NCHUNK, chunk, 0)

        plsc.subcore_barrier()
        pltpu.sync_copy(out_s.at[pl.ds(nb, ROWS_PER_TILE), :],
                        p_hbm.at[cax, pl.ds(nb, ROWS_PER_TILE), :])

    return k(src_flat, dst2, w, den, xp)


def kernel(x, edge_index, batch, W0, as0, ad0, b0, W1, as1, ad1, b1,
           W2, as2, ad2, b2, Wf, bf):
    f32 = jnp.float32
    x_pad = jnp.pad(x.astype(f32), ((0, NP - NNODE), (0, 0)))
    batch2 = jnp.pad(batch, (0, NP - NNODE),
                     constant_values=NGRP).reshape(NP, 1)
    src_flat = edge_index[0]
    dst2 = edge_index[1].reshape(NE // CH, CH)

    w1e = W1[:FEAT] + W1[FEAT:]
    w2e = W2[:FEAT] + W2[FEAT:]
    wfe = (Wf[:FEAT] + Wf[FEAT:]).reshape(FEAT, 1)

    xp, ss, sd = _tc_front0(x_pad, W0, as0.reshape(FEAT, 1),
                            ad0.reshape(FEAT, 1))
    wv, den = _sc_edge_w(src_flat, dst2, ss.reshape(NP), sd.reshape(NP))
    p = _sc_aggregate(src_flat, dst2, wv, den, xp)

    for (we, a_s, a_d, bprev) in ((w1e, as1, ad1, b0), (w2e, as2, ad2, b1)):
        xp, ss, sd = _tc_mid(p, bprev.reshape(1, FEAT), we,
                             a_s.reshape(FEAT, 1), a_d.reshape(FEAT, 1))
        wv, den = _sc_edge_w(src_flat, dst2, ss.reshape(NP), sd.reshape(NP))
        p = _sc_aggregate(src_flat, dst2, wv, den, xp)

    y = _tc_fin(p, b2.reshape(1, FEAT), batch2, wfe, bf.reshape(1, 1))
    return y.reshape(NGRP)


# speedup vs baseline: 16.6730x; 16.6730x over previous
import functools

import jax
import jax.numpy as jnp
from jax import lax
from jax.experimental import pallas as pl
from jax.experimental.pallas import tpu as pltpu
from jax.experimental.pallas import tpu_sc as plsc

NNODE = 10000
NP = 10240
NE = 320000
NEP = 327680
FEAT = 64
PW = 80
NGRP = 64
ROWS_PER_TILE = 640
EW = NEP // 32
CH = 80
NCHUNK = EW // CH
BLK = 640
GRID = NP // BLK


def _tc0_body(x_ref, w_ref, as_ref, ad_ref, xp_ref, ss_ref, sd_ref):
    xp = jnp.dot(x_ref[...], w_ref[...], preferred_element_type=jnp.float32)
    xp_ref[...] = xp
    ss_ref[...] = jnp.dot(xp, as_ref[...], preferred_element_type=jnp.float32)
    sd_ref[...] = jnp.dot(xp, ad_ref[...], preferred_element_type=jnp.float32)


def _tc_front0(x, w, a_s, a_d):
    return pl.pallas_call(
        _tc0_body,
        grid=(GRID,),
        in_specs=[
            pl.BlockSpec((BLK, 128), lambda i: (i, 0)),
            pl.BlockSpec((128, FEAT), lambda i: (0, 0)),
            pl.BlockSpec((FEAT, 1), lambda i: (0, 0)),
            pl.BlockSpec((FEAT, 1), lambda i: (0, 0)),
        ],
        out_specs=[
            pl.BlockSpec((BLK, FEAT), lambda i: (i, 0)),
            pl.BlockSpec((BLK, 1), lambda i: (i, 0)),
            pl.BlockSpec((BLK, 1), lambda i: (i, 0)),
        ],
        out_shape=[
            jax.ShapeDtypeStruct((NP, FEAT), jnp.float32),
            jax.ShapeDtypeStruct((NP, 1), jnp.float32),
            jax.ShapeDtypeStruct((NP, 1), jnp.float32),
        ],
    )(x, w, a_s, a_d)


def _normalize(p_ref, b_ref):
    acc = p_ref[0, :, 0:FEAT] + p_ref[1, :, 0:FEAT]
    den = p_ref[0, :, FEAT:FEAT + 1] + p_ref[1, :, FEAT:FEAT + 1]
    r0 = 1.0 / den
    r0 = r0 * (2.0 - den * r0)
    r0 = r0 * (2.0 - den * r0)
    safe = jnp.where(den > 0.0, r0, 0.0)
    return jnp.maximum(acc * safe + b_ref[...], 0.0)


def _tcmid_body(p_ref, b_ref, w_ref, as_ref, ad_ref, xp_ref, ss_ref, sd_ref):
    r = _normalize(p_ref, b_ref)
    hcat = jnp.concatenate([r, r], axis=1)
    xp = jnp.dot(hcat, w_ref[...], preferred_element_type=jnp.float32)
    xp_ref[...] = xp
    ss_ref[...] = jnp.dot(xp, as_ref[...], preferred_element_type=jnp.float32)
    sd_ref[...] = jnp.dot(xp, ad_ref[...], preferred_element_type=jnp.float32)


def _tc_mid(p, b, w_eff, a_s, a_d):
    return pl.pallas_call(
        _tcmid_body,
        grid=(GRID,),
        in_specs=[
            pl.BlockSpec((2, BLK, PW), lambda i: (0, i, 0)),
            pl.BlockSpec((1, FEAT), lambda i: (0, 0)),
            pl.BlockSpec((2 * FEAT, FEAT), lambda i: (0, 0)),
            pl.BlockSpec((FEAT, 1), lambda i: (0, 0)),
            pl.BlockSpec((FEAT, 1), lambda i: (0, 0)),
        ],
        out_specs=[
            pl.BlockSpec((BLK, FEAT), lambda i: (i, 0)),
            pl.BlockSpec((BLK, 1), lambda i: (i, 0)),
            pl.BlockSpec((BLK, 1), lambda i: (i, 0)),
        ],
        out_shape=[
            jax.ShapeDtypeStruct((NP, FEAT), jnp.float32),
            jax.ShapeDtypeStruct((NP, 1), jnp.float32),
            jax.ShapeDtypeStruct((NP, 1), jnp.float32),
        ],
    )(p, b, w_eff, a_s, a_d)


def _tcfin_body(p_ref, b_ref, bat_ref, wf_ref, bf_ref, y_ref, acc_ref):
    i = pl.program_id(0)
    r = _normalize(p_ref, b_ref)
    grp = bat_ref[...]
    onehot = jnp.where(
        grp == lax.broadcasted_iota(jnp.int32, (BLK, NGRP), 1), 1.0, 0.0)
    part = lax.dot_general(onehot, r, (((0,), (0,)), ((), ())),
                           preferred_element_type=jnp.float32,
                           precision=lax.Precision.HIGHEST)

    @pl.when(i == 0)
    def _():
        acc_ref[...] = part

    @pl.when(i > 0)
    def _():
        acc_ref[...] = acc_ref[...] + part

    @pl.when(i == GRID - 1)
    def _():
        pooled = jnp.concatenate([acc_ref[...], acc_ref[...]], axis=1)
        y_ref[...] = jnp.dot(pooled, wf_ref[...],
                             preferred_element_type=jnp.float32) + bf_ref[...]


def _tc_fin(p, b, batch2, wf_eff, bf):
    return pl.pallas_call(
        _tcfin_body,
        grid=(GRID,),
        in_specs=[
            pl.BlockSpec((2, BLK, PW), lambda i: (0, i, 0)),
            pl.BlockSpec((1, FEAT), lambda i: (0, 0)),
            pl.BlockSpec((BLK, 1), lambda i: (i, 0)),
            pl.BlockSpec((2 * FEAT, 1), lambda i: (0, 0)),
            pl.BlockSpec((1, 1), lambda i: (0, 0)),
        ],
        out_specs=pl.BlockSpec((NGRP, 1), lambda i: (0, 0)),
        out_shape=jax.ShapeDtypeStruct((NGRP, 1), jnp.float32),
        scratch_shapes=[pltpu.VMEM((NGRP, NGRP), jnp.float32)],
    )(p, b, batch2, wf_eff, bf)


_MESH = plsc.VectorSubcoreMesh(core_axis_name="c", subcore_axis_name="s")
_I32 = jnp.int32
_F32 = jnp.float32


def _sc_layer(src_flat, dst2, ss, sd, xp):
    @functools.partial(
        pl.kernel,
        mesh=_MESH,
        compiler_params=pltpu.CompilerParams(
            needs_layout_passes=False, use_tc_tiling_on_sc=False),
        out_type=jax.ShapeDtypeStruct((2, NP, PW), _F32),
        scratch_types=[
            pltpu.VMEM((EW,), _I32),
            pltpu.VMEM((NCHUNK, CH), _I32),
            pltpu.VMEM((NP,), _F32),
            pltpu.VMEM((NP,), _F32),
            pltpu.VMEM((CH,), _F32),
            pltpu.VMEM((CH, FEAT), _F32),
            pltpu.VMEM((CH, PW), _F32),
            pltpu.VMEM((16,), _F32),
            pltpu.VMEM_SHARED((NP, PW), _F32),
        ],
    )
    def k(src_hbm, dst_hbm, ss_hbm, sd_hbm, xp_hbm, p_hbm,
          src_t, dst_c, ss_t, sd_t, w_c, rows, rows_ext, red16, out_s):
        cax = lax.axis_index("c")
        sax = lax.axis_index("s")
        wid = cax * 16 + sax
        nb = sax * ROWS_PER_TILE

        pltpu.sync_copy(src_hbm.at[pl.ds(wid * EW, EW)], src_t)
        pltpu.sync_copy(dst_hbm.at[pl.ds(wid * NCHUNK, NCHUNK), :], dst_c)
        pltpu.sync_copy(ss_hbm, ss_t)
        pltpu.sync_copy(sd_hbm, sd_t)

        zero16 = jnp.zeros((16,), _F32)

        def zrow(j, _):
            for q in range(PW // 16):
                rows_ext[j, pl.ds(q * 16, 16)] = zero16
            return 0

        lax.fori_loop(0, CH, zrow, 0)
        for rep in range(ROWS_PER_TILE // CH):
            pltpu.sync_copy(rows_ext, out_s.at[pl.ds(nb + rep * CH, CH), :])

        neg = jnp.full((16,), -3.0e38, _F32)

        def mx(i, carry):
            a, bb = carry
            sl = pl.ds(i * 16, 16)
            return (jnp.maximum(a, ss_t[sl]), jnp.maximum(bb, sd_t[sl]))

        mss, msd = lax.fori_loop(0, NP // 16, mx, (neg, neg))

        lanes = lax.broadcasted_iota(_I32, (16,), 0)

        def _allmax(v):
            for kk in (8, 4, 2, 1):
                red16[...] = v
                v = jnp.maximum(
                    v, plsc.load_gather(
                        red16,
                        [lax.bitwise_xor(lanes, jnp.full((16,), kk, _I32))]))
            return v

        cvec = jnp.maximum(_allmax(mss) + _allmax(msd), 0.0)
        unit0 = jnp.where(lanes == 0, 1.0, 0.0).astype(_F32)
        zero16i = jnp.zeros((16,), _I32)

        plsc.subcore_barrier()

        def chunk(ci, _):
            eb = ci * CH

            def wvec(i, _):
                fl = pl.ds(eb + i * 16, 16)
                sl = pl.ds(i * 16, 16)
                si = src_t[fl]
                di = dst_c[ci, sl]
                e = (plsc.load_gather(ss_t, [si])
                     + plsc.load_gather(sd_t, [di]))
                e = jnp.where(e > 0.0, e, 0.2 * e)
                w_c[sl] = jnp.exp(e - cvec)
                return 0

            lax.fori_loop(0, CH // 16, wvec, 0)

            pltpu.sync_copy(xp_hbm.at[src_t.at[pl.ds(eb, CH)]], rows)

            def scale(j, _):
                wv = plsc.load_gather(w_c, [zero16i + j])
                for q in range(FEAT // 16):
                    sl = pl.ds(q * 16, 16)
                    rows_ext[j, sl] = rows[j, sl] * wv
                rows_ext[j, pl.ds(FEAT, 16)] = wv * unit0
                return 0

            lax.fori_loop(0, CH, scale, 0)

            pltpu.sync_copy(rows_ext, out_s.at[dst_c.at[ci]], add=True)
            return 0

        lax.fori_loop(0, NCHUNK, chunk, 0)

        plsc.subcore_barrier()
        pltpu.sync_copy(out_s.at[pl.ds(nb, ROWS_PER_TILE), :],
                        p_hbm.at[cax, pl.ds(nb, ROWS_PER_TILE), :])

    return k(src_flat, dst2, ss, sd, xp)


def kernel(x, edge_index, batch, W0, as0, ad0, b0, W1, as1, ad1, b1,
           W2, as2, ad2, b2, Wf, bf):
    f32 = jnp.float32
    x_pad = jnp.pad(x.astype(f32), ((0, NP - NNODE), (0, 0)))
    batch2 = jnp.pad(batch, (0, NP - NNODE),
                     constant_values=NGRP).reshape(NP, 1)
    src_flat = jnp.pad(edge_index[0], (0, NEP - NE), constant_values=NNODE)
    dst2 = jnp.pad(edge_index[1], (0, NEP - NE),
                   constant_values=NNODE).reshape(NEP // CH, CH)

    xp, ss, sd = _tc_front0(x_pad, W0, as0.reshape(FEAT, 1),
                            ad0.reshape(FEAT, 1))
    p = _sc_layer(src_flat, dst2, ss.reshape(NP), sd.reshape(NP), xp)

    for (wfull, a_s, a_d, bprev) in ((W1, as1, ad1, b0), (W2, as2, ad2, b1)):
        xp, ss, sd = _tc_mid(p, bprev.reshape(1, FEAT), wfull,
                             a_s.reshape(FEAT, 1), a_d.reshape(FEAT, 1))
        p = _sc_layer(src_flat, dst2, ss.reshape(NP), sd.reshape(NP), xp)

    y = _tc_fin(p, b2.reshape(1, FEAT), batch2, Wf, bf.reshape(1, 1))
    return y.reshape(NGRP)


# speedup vs baseline: 17.3553x; 1.0409x over previous
import functools

import jax
import jax.numpy as jnp
from jax import lax
from jax.experimental import pallas as pl
from jax.experimental.pallas import tpu as pltpu
from jax.experimental.pallas import tpu_sc as plsc

NNODE = 10000
NP = 10240
NE = 320000
NEP = 327680
FEAT = 64
PW = 80
NGRP = 64
ROWS_PER_TILE = 640
EW = NEP // 32
CH = 128
NCHUNK = EW // CH
BLK = 640
GRID = NP // BLK


def _tc0_body(x_ref, w_ref, as_ref, ad_ref, xp_ref, ss_ref, sd_ref):
    xp = jnp.dot(x_ref[...], w_ref[...], preferred_element_type=jnp.float32)
    xp_ref[...] = xp
    ss_ref[...] = jnp.dot(xp, as_ref[...], preferred_element_type=jnp.float32)
    sd_ref[...] = jnp.dot(xp, ad_ref[...], preferred_element_type=jnp.float32)


def _tc_front0(x, w, a_s, a_d):
    return pl.pallas_call(
        _tc0_body,
        grid=(GRID,),
        in_specs=[
            pl.BlockSpec((BLK, 128), lambda i: (i, 0)),
            pl.BlockSpec((128, FEAT), lambda i: (0, 0)),
            pl.BlockSpec((FEAT, 1), lambda i: (0, 0)),
            pl.BlockSpec((FEAT, 1), lambda i: (0, 0)),
        ],
        out_specs=[
            pl.BlockSpec((BLK, FEAT), lambda i: (i, 0)),
            pl.BlockSpec((BLK, 1), lambda i: (i, 0)),
            pl.BlockSpec((BLK, 1), lambda i: (i, 0)),
        ],
        out_shape=[
            jax.ShapeDtypeStruct((NP, FEAT), jnp.float32),
            jax.ShapeDtypeStruct((NP, 1), jnp.float32),
            jax.ShapeDtypeStruct((NP, 1), jnp.float32),
        ],
    )(x, w, a_s, a_d)


def _normalize(p_ref, b_ref):
    acc = p_ref[0, :, 0:FEAT] + p_ref[1, :, 0:FEAT]
    den = p_ref[0, :, FEAT:FEAT + 1] + p_ref[1, :, FEAT:FEAT + 1]
    r0 = 1.0 / den
    r0 = r0 * (2.0 - den * r0)
    r0 = r0 * (2.0 - den * r0)
    safe = jnp.where(den > 0.0, r0, 0.0)
    return jnp.maximum(acc * safe + b_ref[...], 0.0)


def _tcmid_body(p_ref, b_ref, w_ref, as_ref, ad_ref, xp_ref, ss_ref, sd_ref):
    r = _normalize(p_ref, b_ref)
    hcat = jnp.concatenate([r, r], axis=1)
    xp = jnp.dot(hcat, w_ref[...], preferred_element_type=jnp.float32)
    xp_ref[...] = xp
    ss_ref[...] = jnp.dot(xp, as_ref[...], preferred_element_type=jnp.float32)
    sd_ref[...] = jnp.dot(xp, ad_ref[...], preferred_element_type=jnp.float32)


def _tc_mid(p, b, w_eff, a_s, a_d):
    return pl.pallas_call(
        _tcmid_body,
        grid=(GRID,),
        in_specs=[
            pl.BlockSpec((2, BLK, PW), lambda i: (0, i, 0)),
            pl.BlockSpec((1, FEAT), lambda i: (0, 0)),
            pl.BlockSpec((2 * FEAT, FEAT), lambda i: (0, 0)),
            pl.BlockSpec((FEAT, 1), lambda i: (0, 0)),
            pl.BlockSpec((FEAT, 1), lambda i: (0, 0)),
        ],
        out_specs=[
            pl.BlockSpec((BLK, FEAT), lambda i: (i, 0)),
            pl.BlockSpec((BLK, 1), lambda i: (i, 0)),
            pl.BlockSpec((BLK, 1), lambda i: (i, 0)),
        ],
        out_shape=[
            jax.ShapeDtypeStruct((NP, FEAT), jnp.float32),
            jax.ShapeDtypeStruct((NP, 1), jnp.float32),
            jax.ShapeDtypeStruct((NP, 1), jnp.float32),
        ],
    )(p, b, w_eff, a_s, a_d)


def _tcfin_body(p_ref, b_ref, bat_ref, wf_ref, bf_ref, y_ref, acc_ref):
    i = pl.program_id(0)
    r = _normalize(p_ref, b_ref)
    grp = bat_ref[...]
    onehot = jnp.where(
        grp == lax.broadcasted_iota(jnp.int32, (BLK, NGRP), 1), 1.0, 0.0)
    part = lax.dot_general(onehot, r, (((0,), (0,)), ((), ())),
                           preferred_element_type=jnp.float32,
                           precision=lax.Precision.HIGHEST)

    @pl.when(i == 0)
    def _():
        acc_ref[...] = part

    @pl.when(i > 0)
    def _():
        acc_ref[...] = acc_ref[...] + part

    @pl.when(i == GRID - 1)
    def _():
        pooled = jnp.concatenate([acc_ref[...], acc_ref[...]], axis=1)
        y_ref[...] = jnp.dot(pooled, wf_ref[...],
                             preferred_element_type=jnp.float32) + bf_ref[...]


def _tc_fin(p, b, batch2, wf_eff, bf):
    return pl.pallas_call(
        _tcfin_body,
        grid=(GRID,),
        in_specs=[
            pl.BlockSpec((2, BLK, PW), lambda i: (0, i, 0)),
            pl.BlockSpec((1, FEAT), lambda i: (0, 0)),
            pl.BlockSpec((BLK, 1), lambda i: (i, 0)),
            pl.BlockSpec((2 * FEAT, 1), lambda i: (0, 0)),
            pl.BlockSpec((1, 1), lambda i: (0, 0)),
        ],
        out_specs=pl.BlockSpec((NGRP, 1), lambda i: (0, 0)),
        out_shape=jax.ShapeDtypeStruct((NGRP, 1), jnp.float32),
        scratch_shapes=[pltpu.VMEM((NGRP, NGRP), jnp.float32)],
    )(p, b, batch2, wf_eff, bf)


_MESH = plsc.VectorSubcoreMesh(core_axis_name="c", subcore_axis_name="s")
_I32 = jnp.int32
_F32 = jnp.float32


def _sc_layer(src_flat, dst2, ss, sd, xp):
    @functools.partial(
        pl.kernel,
        mesh=_MESH,
        compiler_params=pltpu.CompilerParams(
            needs_layout_passes=False, use_tc_tiling_on_sc=False),
        out_type=jax.ShapeDtypeStruct((2, NP, PW), _F32),
        scratch_types=[
            pltpu.VMEM((EW,), _I32),
            pltpu.VMEM((NCHUNK, CH), _I32),
            pltpu.VMEM((NP,), _F32),
            pltpu.VMEM((NP,), _F32),
            pltpu.VMEM((CH,), _F32),
            pltpu.VMEM((CH, FEAT), _F32),
            pltpu.VMEM((CH, PW), _F32),
            pltpu.VMEM((16,), _F32),
            pltpu.VMEM_SHARED((NP, PW), _F32),
        ],
    )
    def k(src_hbm, dst_hbm, ss_hbm, sd_hbm, xp_hbm, p_hbm,
          src_t, dst_c, ss_t, sd_t, w_c, rows, rows_ext, red16, out_s):
        cax = lax.axis_index("c")
        sax = lax.axis_index("s")
        wid = cax * 16 + sax
        nb = sax * ROWS_PER_TILE

        pltpu.sync_copy(src_hbm.at[pl.ds(wid * EW, EW)], src_t)
        pltpu.sync_copy(dst_hbm.at[pl.ds(wid * NCHUNK, NCHUNK), :], dst_c)
        pltpu.sync_copy(ss_hbm, ss_t)
        pltpu.sync_copy(sd_hbm, sd_t)

        zero16 = jnp.zeros((16,), _F32)

        def zrow(j, _):
            for q in range(PW // 16):
                rows_ext[j, pl.ds(q * 16, 16)] = zero16
            return 0

        lax.fori_loop(0, CH, zrow, 0)
        for rep in range(ROWS_PER_TILE // CH):
            pltpu.sync_copy(rows_ext, out_s.at[pl.ds(nb + rep * CH, CH), :])

        neg = jnp.full((16,), -3.0e38, _F32)

        def mx(i, carry):
            a, bb = carry
            sl = pl.ds(i * 16, 16)
            return (jnp.maximum(a, ss_t[sl]), jnp.maximum(bb, sd_t[sl]))

        mss, msd = lax.fori_loop(0, NP // 16, mx, (neg, neg))

        lanes = lax.broadcasted_iota(_I32, (16,), 0)

        def _allmax(v):
            for kk in (8, 4, 2, 1):
                red16[...] = v
                v = jnp.maximum(
                    v, plsc.load_gather(
                        red16,
                        [lax.bitwise_xor(lanes, jnp.full((16,), kk, _I32))]))
            return v

        cvec = jnp.maximum(_allmax(mss) + _allmax(msd), 0.0)
        unit0 = jnp.where(lanes == 0, 1.0, 0.0).astype(_F32)
        zero16i = jnp.zeros((16,), _I32)

        plsc.subcore_barrier()

        def chunk(ci, _):
            eb = ci * CH

            def wvec(i, _):
                fl = pl.ds(eb + i * 16, 16)
                sl = pl.ds(i * 16, 16)
                si = src_t[fl]
                di = dst_c[ci, sl]
                e = (plsc.load_gather(ss_t, [si])
                     + plsc.load_gather(sd_t, [di]))
                e = jnp.where(e > 0.0, e, 0.2 * e)
                w_c[sl] = jnp.exp(e - cvec)
                return 0

            lax.fori_loop(0, CH // 16, wvec, 0)

            pltpu.sync_copy(xp_hbm.at[src_t.at[pl.ds(eb, CH)]], rows)

            def scale(j, _):
                wv = plsc.load_gather(w_c, [zero16i + j])
                for q in range(FEAT // 16):
                    sl = pl.ds(q * 16, 16)
                    rows_ext[j, sl] = rows[j, sl] * wv
                rows_ext[j, pl.ds(FEAT, 16)] = wv * unit0
                return 0

            lax.fori_loop(0, CH, scale, 0)

            pltpu.sync_copy(rows_ext, out_s.at[dst_c.at[ci]], add=True)
            return 0

        lax.fori_loop(0, NCHUNK, chunk, 0)

        plsc.subcore_barrier()
        pltpu.sync_copy(out_s.at[pl.ds(nb, ROWS_PER_TILE), :],
                        p_hbm.at[cax, pl.ds(nb, ROWS_PER_TILE), :])

    return k(src_flat, dst2, ss, sd, xp)


def kernel(x, edge_index, batch, W0, as0, ad0, b0, W1, as1, ad1, b1,
           W2, as2, ad2, b2, Wf, bf):
    f32 = jnp.float32
    x_pad = jnp.pad(x.astype(f32), ((0, NP - NNODE), (0, 0)))
    batch2 = jnp.pad(batch, (0, NP - NNODE),
                     constant_values=NGRP).reshape(NP, 1)
    src_flat = jnp.pad(edge_index[0], (0, NEP - NE), constant_values=NNODE)
    dst2 = jnp.pad(edge_index[1], (0, NEP - NE),
                   constant_values=NNODE).reshape(NEP // CH, CH)

    xp, ss, sd = _tc_front0(x_pad, W0, as0.reshape(FEAT, 1),
                            ad0.reshape(FEAT, 1))
    p = _sc_layer(src_flat, dst2, ss.reshape(NP), sd.reshape(NP), xp)

    for (wfull, a_s, a_d, bprev) in ((W1, as1, ad1, b0), (W2, as2, ad2, b1)):
        xp, ss, sd = _tc_mid(p, bprev.reshape(1, FEAT), wfull,
                             a_s.reshape(FEAT, 1), a_d.reshape(FEAT, 1))
        p = _sc_layer(src_flat, dst2, ss.reshape(NP), sd.reshape(NP), xp)

    y = _tc_fin(p, b2.reshape(1, FEAT), batch2, Wf, bf.reshape(1, 1))
    return y.reshape(NGRP)


# speedup vs baseline: 18.7721x; 1.0816x over previous
import functools

import jax
import jax.numpy as jnp
from jax import lax
from jax.experimental import pallas as pl
from jax.experimental.pallas import tpu as pltpu
from jax.experimental.pallas import tpu_sc as plsc

NNODE = 10000
NP = 10240
NE = 320000
NEP = 327680
FEAT = 64
PW = 80
NGRP = 64
ROWS_PER_TILE = 640
EW = NEP // 32
CH = 128
NCHUNK = EW // CH
BLK = 640
GRID = NP // BLK


def _tc0_body(x_ref, w_ref, as_ref, ad_ref, xp_ref, ss_ref, sd_ref):
    xp = jnp.dot(x_ref[...], w_ref[...], preferred_element_type=jnp.float32)
    xp_ref[...] = xp
    ss_ref[...] = jnp.dot(xp, as_ref[...], preferred_element_type=jnp.float32)
    sd_ref[...] = jnp.dot(xp, ad_ref[...], preferred_element_type=jnp.float32)


def _tc_front0(x, w, a_s, a_d):
    return pl.pallas_call(
        _tc0_body,
        grid=(GRID,),
        in_specs=[
            pl.BlockSpec((BLK, 128), lambda i: (i, 0)),
            pl.BlockSpec((128, FEAT), lambda i: (0, 0)),
            pl.BlockSpec((FEAT, 1), lambda i: (0, 0)),
            pl.BlockSpec((FEAT, 1), lambda i: (0, 0)),
        ],
        out_specs=[
            pl.BlockSpec((BLK, FEAT), lambda i: (i, 0)),
            pl.BlockSpec((BLK, 1), lambda i: (i, 0)),
            pl.BlockSpec((BLK, 1), lambda i: (i, 0)),
        ],
        out_shape=[
            jax.ShapeDtypeStruct((NP, FEAT), jnp.float32),
            jax.ShapeDtypeStruct((NP, 1), jnp.float32),
            jax.ShapeDtypeStruct((NP, 1), jnp.float32),
        ],
    )(x, w, a_s, a_d)


def _normalize(p_ref, b_ref):
    acc = p_ref[0, :, 0:FEAT] + p_ref[1, :, 0:FEAT]
    den = p_ref[0, :, FEAT:FEAT + 1] + p_ref[1, :, FEAT:FEAT + 1]
    r0 = 1.0 / den
    r0 = r0 * (2.0 - den * r0)
    r0 = r0 * (2.0 - den * r0)
    safe = jnp.where(den > 0.0, r0, 0.0)
    return jnp.maximum(acc * safe + b_ref[...], 0.0)


def _tcmid_body(p_ref, b_ref, w_ref, as_ref, ad_ref, xp_ref, ss_ref, sd_ref):
    r = _normalize(p_ref, b_ref)
    hcat = jnp.concatenate([r, r], axis=1)
    xp = jnp.dot(hcat, w_ref[...], preferred_element_type=jnp.float32)
    xp_ref[...] = xp
    ss_ref[...] = jnp.dot(xp, as_ref[...], preferred_element_type=jnp.float32)
    sd_ref[...] = jnp.dot(xp, ad_ref[...], preferred_element_type=jnp.float32)


def _tc_mid(p, b, w_eff, a_s, a_d):
    return pl.pallas_call(
        _tcmid_body,
        grid=(GRID,),
        in_specs=[
            pl.BlockSpec((2, BLK, PW), lambda i: (0, i, 0)),
            pl.BlockSpec((1, FEAT), lambda i: (0, 0)),
            pl.BlockSpec((2 * FEAT, FEAT), lambda i: (0, 0)),
            pl.BlockSpec((FEAT, 1), lambda i: (0, 0)),
            pl.BlockSpec((FEAT, 1), lambda i: (0, 0)),
        ],
        out_specs=[
            pl.BlockSpec((BLK, FEAT), lambda i: (i, 0)),
            pl.BlockSpec((BLK, 1), lambda i: (i, 0)),
            pl.BlockSpec((BLK, 1), lambda i: (i, 0)),
        ],
        out_shape=[
            jax.ShapeDtypeStruct((NP, FEAT), jnp.float32),
            jax.ShapeDtypeStruct((NP, 1), jnp.float32),
            jax.ShapeDtypeStruct((NP, 1), jnp.float32),
        ],
    )(p, b, w_eff, a_s, a_d)


def _tcfin_body(p_ref, b_ref, bat_ref, wf_ref, bf_ref, y_ref, acc_ref):
    i = pl.program_id(0)
    r = _normalize(p_ref, b_ref)
    grp = bat_ref[...]
    onehot = jnp.where(
        grp == lax.broadcasted_iota(jnp.int32, (BLK, NGRP), 1), 1.0, 0.0)
    part = lax.dot_general(onehot, r, (((0,), (0,)), ((), ())),
                           preferred_element_type=jnp.float32,
                           precision=lax.Precision.HIGHEST)

    @pl.when(i == 0)
    def _():
        acc_ref[...] = part

    @pl.when(i > 0)
    def _():
        acc_ref[...] = acc_ref[...] + part

    @pl.when(i == GRID - 1)
    def _():
        pooled = jnp.concatenate([acc_ref[...], acc_ref[...]], axis=1)
        y_ref[...] = jnp.dot(pooled, wf_ref[...],
                             preferred_element_type=jnp.float32) + bf_ref[...]


def _tc_fin(p, b, batch2, wf_eff, bf):
    return pl.pallas_call(
        _tcfin_body,
        grid=(GRID,),
        in_specs=[
            pl.BlockSpec((2, BLK, PW), lambda i: (0, i, 0)),
            pl.BlockSpec((1, FEAT), lambda i: (0, 0)),
            pl.BlockSpec((BLK, 1), lambda i: (i, 0)),
            pl.BlockSpec((2 * FEAT, 1), lambda i: (0, 0)),
            pl.BlockSpec((1, 1), lambda i: (0, 0)),
        ],
        out_specs=pl.BlockSpec((NGRP, 1), lambda i: (0, 0)),
        out_shape=jax.ShapeDtypeStruct((NGRP, 1), jnp.float32),
        scratch_shapes=[pltpu.VMEM((NGRP, NGRP), jnp.float32)],
    )(p, b, batch2, wf_eff, bf)


_MESH = plsc.VectorSubcoreMesh(core_axis_name="c", subcore_axis_name="s")
_I32 = jnp.int32
_F32 = jnp.float32


def _sc_layer(src_flat, dst2, ss, sd, xp):
    @functools.partial(
        pl.kernel,
        mesh=_MESH,
        compiler_params=pltpu.CompilerParams(
            needs_layout_passes=False, use_tc_tiling_on_sc=False),
        out_type=jax.ShapeDtypeStruct((2, NP, PW), _F32),
        scratch_types=[
            pltpu.VMEM((EW,), _I32),
            pltpu.VMEM((NCHUNK, CH), _I32),
            pltpu.VMEM((NP,), _F32),
            pltpu.VMEM((NP,), _F32),
            pltpu.VMEM((CH,), _F32),
            pltpu.VMEM((CH, FEAT), _F32),
            pltpu.VMEM((CH, PW), _F32),
            pltpu.VMEM((16,), _F32),
            pltpu.VMEM_SHARED((NP, PW), _F32),
            pltpu.SemaphoreType.DMA,
            pltpu.SemaphoreType.DMA,
        ],
    )
    def k(src_hbm, dst_hbm, ss_hbm, sd_hbm, xp_hbm, p_hbm,
          src_t, dst_c, ss_t, sd_t, w_c, rows, rows_ext, red16, out_s,
          gsem, ssem):
        cax = lax.axis_index("c")
        sax = lax.axis_index("s")
        wid = cax * 16 + sax
        nb = sax * ROWS_PER_TILE

        pltpu.sync_copy(src_hbm.at[pl.ds(wid * EW, EW)], src_t)
        pltpu.sync_copy(dst_hbm.at[pl.ds(wid * NCHUNK, NCHUNK), :], dst_c)
        pltpu.sync_copy(ss_hbm, ss_t)
        pltpu.sync_copy(sd_hbm, sd_t)

        zero16 = jnp.zeros((16,), _F32)

        def zrow(j, _):
            for q in range(PW // 16):
                rows_ext[j, pl.ds(q * 16, 16)] = zero16
            return 0

        lax.fori_loop(0, CH, zrow, 0)
        for rep in range(ROWS_PER_TILE // CH):
            pltpu.sync_copy(rows_ext, out_s.at[pl.ds(nb + rep * CH, CH), :])

        neg = jnp.full((16,), -3.0e38, _F32)

        def mx(i, carry):
            a, bb = carry
            sl = pl.ds(i * 16, 16)
            return (jnp.maximum(a, ss_t[sl]), jnp.maximum(bb, sd_t[sl]))

        mss, msd = lax.fori_loop(0, NP // 16, mx, (neg, neg))

        lanes = lax.broadcasted_iota(_I32, (16,), 0)

        def _allmax(v):
            for kk in (8, 4, 2, 1):
                red16[...] = v
                v = jnp.maximum(
                    v, plsc.load_gather(
                        red16,
                        [lax.bitwise_xor(lanes, jnp.full((16,), kk, _I32))]))
            return v

        cvec = jnp.maximum(_allmax(mss) + _allmax(msd), 0.0)
        unit0 = jnp.where(lanes == 0, 1.0, 0.0).astype(_F32)
        zero16i = jnp.zeros((16,), _I32)

        plsc.subcore_barrier()

        drain_src = p_hbm.at[0, pl.ds(0, CH), :]

        def chunk(ci, _):
            eb = ci * CH
            cpg = pltpu.async_copy(xp_hbm.at[src_t.at[pl.ds(eb, CH)]],
                                   rows, gsem)

            def wvec(i, _):
                fl = pl.ds(eb + i * 16, 16)
                sl = pl.ds(i * 16, 16)
                si = src_t[fl]
                di = dst_c[ci, sl]
                e = (plsc.load_gather(ss_t, [si])
                     + plsc.load_gather(sd_t, [di]))
                e = jnp.where(e > 0.0, e, 0.2 * e)
                w_c[sl] = jnp.exp(e - cvec)
                return 0

            lax.fori_loop(0, CH // 16, wvec, 0)
            cpg.wait()

            @pl.when(ci > 0)
            def _():
                pltpu.make_async_copy(drain_src, rows_ext, ssem).wait()

            def scale(j, _):
                wv = plsc.load_gather(w_c, [zero16i + j])
                for q in range(FEAT // 16):
                    sl = pl.ds(q * 16, 16)
                    rows_ext[j, sl] = rows[j, sl] * wv
                rows_ext[j, pl.ds(FEAT, 16)] = wv * unit0
                return 0

            lax.fori_loop(0, CH, scale, 0)

            pltpu.async_copy(rows_ext, out_s.at[dst_c.at[ci]], ssem, add=True)
            return 0

        lax.fori_loop(0, NCHUNK, chunk, 0)
        pltpu.make_async_copy(drain_src, rows_ext, ssem).wait()

        plsc.subcore_barrier()
        pltpu.sync_copy(out_s.at[pl.ds(nb, ROWS_PER_TILE), :],
                        p_hbm.at[cax, pl.ds(nb, ROWS_PER_TILE), :])

    return k(src_flat, dst2, ss, sd, xp)


def kernel(x, edge_index, batch, W0, as0, ad0, b0, W1, as1, ad1, b1,
           W2, as2, ad2, b2, Wf, bf):
    f32 = jnp.float32
    x_pad = jnp.pad(x.astype(f32), ((0, NP - NNODE), (0, 0)))
    batch2 = jnp.pad(batch, (0, NP - NNODE),
                     constant_values=NGRP).reshape(NP, 1)
    src_flat = jnp.pad(edge_index[0], (0, NEP - NE), constant_values=NNODE)
    dst2 = jnp.pad(edge_index[1], (0, NEP - NE),
                   constant_values=NNODE).reshape(NEP // CH, CH)

    xp, ss, sd = _tc_front0(x_pad, W0, as0.reshape(FEAT, 1),
                            ad0.reshape(FEAT, 1))
    p = _sc_layer(src_flat, dst2, ss.reshape(NP), sd.reshape(NP), xp)

    for (wfull, a_s, a_d, bprev) in ((W1, as1, ad1, b0), (W2, as2, ad2, b1)):
        xp, ss, sd = _tc_mid(p, bprev.reshape(1, FEAT), wfull,
                             a_s.reshape(FEAT, 1), a_d.reshape(FEAT, 1))
        p = _sc_layer(src_flat, dst2, ss.reshape(NP), sd.reshape(NP), xp)

    y = _tc_fin(p, b2.reshape(1, FEAT), batch2, Wf, bf.reshape(1, 1))
    return y.reshape(NGRP)
